# baseline (speedup 1.0000x reference)
import functools

import jax
import jax.numpy as jnp
from jax import lax
from jax.experimental import pallas as pl
from jax.experimental.pallas import tpu as pltpu
from jax.experimental.pallas import tpu_sc as plsc

VOCAB = 100000
HIDDEN = 768
MAX_POS = 2048
BATCH = 4
SEQ = 2048
EPS = 1e-12

NC = 2
NS = 16
NW = NC * NS
TOKENS = BATCH * SEQ

PS = ((0, 1280), (1280, 768))
PIECES = len(PS)
LNB = 256
NCHUNK = 2


def _sc_gather_body(start, size, idx_hbm, word_hbm, out_hbm,
                    idx_v, buf0, buf1, sem0, sem1):
    tpw = size // 8
    cw = tpw // NCHUNK
    wid = lax.axis_index("s") * NC + lax.axis_index("c")
    base = (wid // 8) * SEQ + start + tpw * (wid % 8)
    pltpu.sync_copy(idx_hbm.at[pl.ds(base, tpw)], idx_v)
    bufs = (buf0, buf1)
    sems = (sem0, sem1)
    cps = [None] * NCHUNK
    cps[0] = pltpu.async_copy(
        word_hbm.at[idx_v.at[pl.ds(0, cw)]], bufs[0], sems[0])
    for c in range(NCHUNK):
        if c + 1 < NCHUNK:
            cps[c + 1] = pltpu.async_copy(
                word_hbm.at[idx_v.at[pl.ds((c + 1) * cw, cw)]],
                bufs[(c + 1) % 2], sems[(c + 1) % 2])
        cps[c].wait()
        pltpu.sync_copy(bufs[c % 2],
                        out_hbm.at[pl.ds(wid * tpw + c * cw, cw)])


def _ln_first_body(g_ref, pos_ref, tt_ref, gam_ref, bet_ref, o_ref):
    x = g_ref[...] + pos_ref[...] + tt_ref[...]
    mean = jnp.mean(x, axis=1, keepdims=True)
    xc = x - mean
    var = jnp.mean(xc * xc, axis=1, keepdims=True)
    o_ref[...] = xc * lax.rsqrt(var + EPS) * gam_ref[...] + bet_ref[...]


def _ln_chain_body(g_ref, pos_ref, tt_ref, gam_ref, bet_ref, prev_ref,
                   o_ref):
    del prev_ref
    _ln_first_body(g_ref, pos_ref, tt_ref, gam_ref, bet_ref, o_ref)


def _make_ln_call(p):
    start, size = PS[p]
    nblk = size // LNB
    s0 = start // LNB
    specs = [
        pl.BlockSpec((LNB, HIDDEN), lambda i, b: (b * nblk + i, 0)),
        pl.BlockSpec((LNB, HIDDEN), lambda i, b: (s0 + i, 0)),
        pl.BlockSpec((1, HIDDEN), lambda i, b: (0, 0)),
        pl.BlockSpec((1, HIDDEN), lambda i, b: (0, 0)),
        pl.BlockSpec((1, HIDDEN), lambda i, b: (0, 0)),
    ]
    out_spec = pl.BlockSpec(
        (LNB, HIDDEN), lambda i, b: (b * (SEQ // LNB) + s0 + i, 0))
    out_shape = jax.ShapeDtypeStruct((TOKENS, HIDDEN), jnp.float32)
    if p == 0:
        return pl.pallas_call(
            _ln_first_body, grid=(nblk, BATCH), in_specs=specs,
            out_specs=out_spec, out_shape=out_shape)
    return pl.pallas_call(
        _ln_chain_body, grid=(nblk, BATCH),
        in_specs=specs + [pl.BlockSpec(memory_space=pltpu.MemorySpace.HBM)],
        out_specs=out_spec, out_shape=out_shape,
        input_output_aliases={5: 0})


@jax.jit
def _bert_embed(input_ids, word_embeddings, position_embeddings,
                token_type_embeddings, ln_gamma, ln_beta):
    idx_flat = input_ids.astype(jnp.int32).reshape(TOKENS)
    tt_row = token_type_embeddings[0].reshape(1, HIDDEN)
    gam2 = ln_gamma.reshape(1, HIDDEN)
    bet2 = ln_beta.reshape(1, HIDDEN)
    gs = []
    for start, size in PS:
        tpw = size // 8
        cw = tpw // NCHUNK
        sc_gather = pl.kernel(
            functools.partial(_sc_gather_body, start, size),
            out_type=jax.ShapeDtypeStruct((BATCH * size, HIDDEN),
                                          jnp.float32),
            mesh=plsc.VectorSubcoreMesh(
                core_axis_name="c", subcore_axis_name="s"),
            scratch_types=[
                pltpu.VMEM((tpw,), jnp.int32),
                pltpu.VMEM((cw, HIDDEN), jnp.float32),
                pltpu.VMEM((cw, HIDDEN), jnp.float32),
                pltpu.SemaphoreType.DMA,
                pltpu.SemaphoreType.DMA,
            ],
        )
        gs.append(sc_gather(idx_flat, word_embeddings))
    out = _make_ln_call(0)(gs[0], position_embeddings, tt_row, gam2, bet2)
    for p in range(1, PIECES):
        out = _make_ln_call(p)(
            gs[p], position_embeddings, tt_row, gam2, bet2, out)
    return out.reshape(BATCH, SEQ, HIDDEN)


def kernel(input_ids, word_embeddings, position_embeddings,
           token_type_embeddings, ln_gamma, ln_beta):
    return _bert_embed(input_ids, word_embeddings, position_embeddings,
                       token_type_embeddings, ln_gamma, ln_beta)


# speedup vs baseline: 1.1912x; 1.1912x over previous
import functools

import jax
import jax.numpy as jnp
from jax import lax
from jax.experimental import pallas as pl
from jax.experimental.pallas import tpu as pltpu
from jax.experimental.pallas import tpu_sc as plsc

VOCAB = 100000
HIDDEN = 768
MAX_POS = 2048
BATCH = 4
SEQ = 2048
EPS = 1e-12

NC = 2
NS = 16
NW = NC * NS
TOKENS = BATCH * SEQ

PS = ((0, 1024), (1024, 1024))
PIECES = len(PS)
LNB = 1024
NCHUNK = 2


def _sc_gather_body(start, size, idx_hbm, word_hbm, out_hbm,
                    idx_v, buf0, buf1, sem0, sem1):
    tpw = size // 8
    cw = tpw // NCHUNK
    wid = lax.axis_index("s") * NC + lax.axis_index("c")
    base = (wid // 8) * SEQ + start + tpw * (wid % 8)
    pltpu.sync_copy(idx_hbm.at[pl.ds(base, tpw)], idx_v)
    bufs = (buf0, buf1)
    sems = (sem0, sem1)
    cps = [None] * NCHUNK
    cps[0] = pltpu.async_copy(
        word_hbm.at[idx_v.at[pl.ds(0, cw)]], bufs[0], sems[0])
    for c in range(NCHUNK):
        if c + 1 < NCHUNK:
            cps[c + 1] = pltpu.async_copy(
                word_hbm.at[idx_v.at[pl.ds((c + 1) * cw, cw)]],
                bufs[(c + 1) % 2], sems[(c + 1) % 2])
        cps[c].wait()
        pltpu.sync_copy(bufs[c % 2],
                        out_hbm.at[pl.ds(wid * tpw + c * cw, cw)])


def _ln_first_body(g_ref, pos_ref, tt_ref, gam_ref, bet_ref, o_ref):
    x = g_ref[...] + pos_ref[...] + tt_ref[...]
    mean = jnp.mean(x, axis=1, keepdims=True)
    xc = x - mean
    var = jnp.mean(xc * xc, axis=1, keepdims=True)
    o_ref[...] = xc * lax.rsqrt(var + EPS) * gam_ref[...] + bet_ref[...]


def _ln_chain_body(g_ref, pos_ref, tt_ref, gam_ref, bet_ref, prev_ref,
                   o_ref):
    del prev_ref
    _ln_first_body(g_ref, pos_ref, tt_ref, gam_ref, bet_ref, o_ref)


def _make_ln_call(p):
    start, size = PS[p]
    nblk = size // LNB
    s0 = start // LNB
    specs = [
        pl.BlockSpec((LNB, HIDDEN), lambda i, b: (b * nblk + i, 0)),
        pl.BlockSpec((LNB, HIDDEN), lambda i, b: (s0 + i, 0)),
        pl.BlockSpec((1, HIDDEN), lambda i, b: (0, 0)),
        pl.BlockSpec((1, HIDDEN), lambda i, b: (0, 0)),
        pl.BlockSpec((1, HIDDEN), lambda i, b: (0, 0)),
    ]
    out_spec = pl.BlockSpec(
        (LNB, HIDDEN), lambda i, b: (b * (SEQ // LNB) + s0 + i, 0))
    out_shape = jax.ShapeDtypeStruct((TOKENS, HIDDEN), jnp.float32)
    if p == 0:
        return pl.pallas_call(
            _ln_first_body, grid=(nblk, BATCH), in_specs=specs,
            out_specs=out_spec, out_shape=out_shape)
    return pl.pallas_call(
        _ln_chain_body, grid=(nblk, BATCH),
        in_specs=specs + [pl.BlockSpec(memory_space=pltpu.MemorySpace.HBM)],
        out_specs=out_spec, out_shape=out_shape,
        input_output_aliases={5: 0})


@jax.jit
def _bert_embed(input_ids, word_embeddings, position_embeddings,
                token_type_embeddings, ln_gamma, ln_beta):
    idx_flat = input_ids.astype(jnp.int32).reshape(TOKENS)
    tt_row = token_type_embeddings[0].reshape(1, HIDDEN)
    gam2 = ln_gamma.reshape(1, HIDDEN)
    bet2 = ln_beta.reshape(1, HIDDEN)
    gs = []
    for start, size in PS:
        tpw = size // 8
        cw = tpw // NCHUNK
        sc_gather = pl.kernel(
            functools.partial(_sc_gather_body, start, size),
            out_type=jax.ShapeDtypeStruct((BATCH * size, HIDDEN),
                                          jnp.float32),
            mesh=plsc.VectorSubcoreMesh(
                core_axis_name="c", subcore_axis_name="s"),
            scratch_types=[
                pltpu.VMEM((tpw,), jnp.int32),
                pltpu.VMEM((cw, HIDDEN), jnp.float32),
                pltpu.VMEM((cw, HIDDEN), jnp.float32),
                pltpu.SemaphoreType.DMA,
                pltpu.SemaphoreType.DMA,
            ],
        )
        gs.append(sc_gather(idx_flat, word_embeddings))
    out = _make_ln_call(0)(gs[0], position_embeddings, tt_row, gam2, bet2)
    for p in range(1, PIECES):
        out = _make_ln_call(p)(
            gs[p], position_embeddings, tt_row, gam2, bet2, out)
    return out.reshape(BATCH, SEQ, HIDDEN)


def kernel(input_ids, word_embeddings, position_embeddings,
           token_type_embeddings, ln_gamma, ln_beta):
    return _bert_embed(input_ids, word_embeddings, position_embeddings,
                       token_type_embeddings, ln_gamma, ln_beta)
